# Initial kernel scaffold; baseline (speedup 1.0000x reference)
import jax
import jax.numpy as jnp
from jax.experimental import pallas as pl


def kernel(inputs, lengths, label_len, W1, b1, W2, W3, W0):
    raise NotImplementedError("write your pallas kernel here")


# speedup vs baseline: 9.4687x; 9.4687x over previous
import functools

import jax
import jax.numpy as jnp
from jax import lax
from jax.experimental import pallas as pl
from jax.experimental.pallas import tpu as pltpu

T, B, D, H, LL = 2048, 8, 512, 512, 4


def _pass1_body(x_ref, tb_ref, s_ref, g_ref):
    pid = pl.program_id(0)
    tt = x_ref.shape[0]

    @pl.when(pid == 0)
    def _init():
        s_ref[...] = jnp.zeros_like(s_ref)
        g_ref[...] = jnp.zeros_like(g_ref)

    x = x_ref[...]
    tvec = lax.broadcasted_iota(jnp.int32, (tt, B, 1), 0) + pid * tt
    tb = tb_ref[...]
    for j in range(LL):
        tbj = tb[j][None]
        le = (tvec <= tbj).astype(jnp.float32)
        eq = (tvec == tbj).astype(jnp.float32)
        s_ref[j] = s_ref[j] + jnp.sum(x * le, axis=0)
        g_ref[j] = g_ref[j] + jnp.sum(x * eq, axis=0)


def _pass2_body(x_ref, w1t_ref, b1_ref, w0_ref, w2t_ref, w3t_ref,
                s_ref, g_ref, tb_ref, inv_ref, out_ref, c_s, x2_s):
    pid = pl.program_id(0)
    nt = pl.num_programs(0)
    tt = x_ref.shape[0]

    @pl.when(pid == 0)
    def _prologue():
        g2 = jnp.dot(g_ref[...].reshape(LL * B, D), w2t_ref[...],
                     preferred_element_type=jnp.float32)
        s3 = jnp.dot(s_ref[...].reshape(LL * B, D), w3t_ref[...],
                     preferred_element_type=jnp.float32)
        inv = inv_ref[...].reshape(LL * B, 1)
        x2_s[...] = g2.reshape(LL, B, H)
        c_s[...] = (g2 + s3 * inv).reshape(LL, B, H)
        out_ref[...] = jnp.zeros_like(out_ref)

    x = x_ref[...]
    x1 = jnp.dot(x.reshape(tt * B, D), w1t_ref[...],
                 preferred_element_type=jnp.float32) + b1_ref[...]
    x1 = x1.reshape(tt, B, H)
    w0 = w0_ref[...].reshape(1, 1, H)
    tvec = lax.broadcasted_iota(jnp.int32, (tt, B, 1), 0) + pid * tt
    tb = tb_ref[...]
    for j in range(LL):
        cj = c_s[j][None]
        sg = jax.nn.sigmoid(x1 + cj)
        score = jnp.sum(sg * w0, axis=-1, keepdims=True)
        le = (tvec <= tb[j][None]).astype(jnp.float32)
        out_ref[j] = out_ref[j] + jnp.sum(x1 * (score * le), axis=0)

    @pl.when(pid == nt - 1)
    def _epilogue():
        out_ref[...] = out_ref[...] + x2_s[...]


@jax.jit
def kernel(inputs, lengths, label_len, W1, b1, W2, W3, W0):
    tt1, tt2 = 512, 256
    tb_i = lengths[None, :].astype(jnp.int32) - label_len + jnp.arange(LL)[:, None]
    tb = tb_i.reshape(LL, B, 1)
    inv = 1.0 / (tb.astype(jnp.float32) + 1.0)

    s_g = pl.pallas_call(
        _pass1_body,
        grid=(T // tt1,),
        in_specs=[
            pl.BlockSpec((tt1, B, D), lambda i: (i, 0, 0)),
            pl.BlockSpec((LL, B, 1), lambda i: (0, 0, 0)),
        ],
        out_specs=[
            pl.BlockSpec((LL, B, D), lambda i: (0, 0, 0)),
            pl.BlockSpec((LL, B, D), lambda i: (0, 0, 0)),
        ],
        out_shape=[
            jax.ShapeDtypeStruct((LL, B, D), jnp.float32),
            jax.ShapeDtypeStruct((LL, B, D), jnp.float32),
        ],
    )(inputs, tb)
    S, G = s_g

    out = pl.pallas_call(
        _pass2_body,
        grid=(T // tt2,),
        in_specs=[
            pl.BlockSpec((tt2, B, D), lambda i: (i, 0, 0)),
            pl.BlockSpec((D, H), lambda i: (0, 0)),
            pl.BlockSpec((1, H), lambda i: (0, 0)),
            pl.BlockSpec((1, H), lambda i: (0, 0)),
            pl.BlockSpec((D, H), lambda i: (0, 0)),
            pl.BlockSpec((D, H), lambda i: (0, 0)),
            pl.BlockSpec((LL, B, D), lambda i: (0, 0, 0)),
            pl.BlockSpec((LL, B, D), lambda i: (0, 0, 0)),
            pl.BlockSpec((LL, B, 1), lambda i: (0, 0, 0)),
            pl.BlockSpec((LL, B, 1), lambda i: (0, 0, 0)),
        ],
        out_specs=pl.BlockSpec((LL, B, H), lambda i: (0, 0, 0)),
        out_shape=jax.ShapeDtypeStruct((LL, B, H), jnp.float32),
        scratch_shapes=[
            pltpu.VMEM((LL, B, H), jnp.float32),
            pltpu.VMEM((LL, B, H), jnp.float32),
        ],
    )(inputs, W1.T, b1.reshape(1, H), W0.reshape(1, H), W2.T, W3.T,
      S, G, tb, inv)

    return jnp.transpose(out, (1, 0, 2))


# speedup vs baseline: 10.5590x; 1.1151x over previous
import jax
import jax.numpy as jnp
from jax import lax
from jax.experimental import pallas as pl
from jax.experimental.pallas import tpu as pltpu

T, B, D, H, LL = 2048, 8, 512, 512, 4
TT = 256
NT = T // TT


def _body(x_ref, w1t_ref, b1_ref, w0_ref, w2t_ref, w3t_ref, tb_v_ref,
          inv_ref, tb_s_ref, out_ref, xc_s, sfull_s, g_s, c_s, x2_s):
    p = pl.program_id(0)
    i = pl.program_id(1)
    start = i * TT
    t_max = tb_s_ref[LL - 1, 0]
    live = start <= t_max

    @pl.when(p == 0)
    def _phase0():
        @pl.when(i == 0)
        def _init():
            sfull_s[...] = jnp.zeros_like(sfull_s)
            g_s[...] = jnp.zeros_like(g_s)

        @pl.when(live)
        def _accum():
            x = x_ref[...]
            xc_s[pl.ds(start * B, TT * B)] = x.astype(jnp.bfloat16).reshape(TT * B, D)
            sfull_s[...] = sfull_s[...] + jnp.sum(x, axis=0)
            tvec = lax.broadcasted_iota(jnp.int32, (TT, B, 1), 0) + start
            tb = tb_v_ref[...]
            for j in range(LL):
                hit = jnp.logical_and(tb_s_ref[j, 0] >= start,
                                      tb_s_ref[j, B - 1] < start + TT)

                @pl.when(hit)
                def _g():
                    eq = (tvec == tb[j][None]).astype(jnp.float32)
                    g_s[j] = g_s[j] + jnp.sum(x * eq, axis=0)

    @pl.when(p == 1)
    def _phase1():
        @pl.when(i == 0)
        def _prologue():
            g = g_s[...]
            s3 = sfull_s[...][None]
            s2 = s3 - g[3][None]
            s1 = s2 - g[2][None]
            s0 = s1 - g[1][None]
            s_all = jnp.concatenate([s0, s1, s2, s3], axis=0)
            g2 = jnp.dot(g.reshape(LL * B, D), w2t_ref[...],
                         preferred_element_type=jnp.float32)
            sw = jnp.dot(s_all.reshape(LL * B, D), w3t_ref[...],
                         preferred_element_type=jnp.float32)
            inv = inv_ref[...].reshape(LL * B, 1)
            x2_s[...] = g2.reshape(LL, B, H)
            c_s[...] = (g2 + sw * inv).reshape(LL, B, H)
            out_ref[...] = jnp.zeros_like(out_ref)

        @pl.when(live)
        def _attend():
            xb = xc_s[pl.ds(start * B, TT * B)]
            x1 = jnp.dot(xb, w1t_ref[...],
                         preferred_element_type=jnp.float32) + b1_ref[...]
            x1 = x1.reshape(TT, B, H)
            w0 = w0_ref[...].reshape(1, 1, H)
            tvec = lax.broadcasted_iota(jnp.int32, (TT, B, 1), 0) + start
            tb = tb_v_ref[...]
            for j in range(LL):
                @pl.when(start <= tb_s_ref[j, 0])
                def _att_j():
                    cj = c_s[j][None]
                    sg = jax.nn.sigmoid(x1 + cj)
                    score = jnp.sum(sg * w0, axis=-1, keepdims=True)
                    le = (tvec <= tb[j][None]).astype(jnp.float32)
                    out_ref[j] = out_ref[j] + jnp.sum(x1 * (score * le),
                                                      axis=0)

        @pl.when(i == NT - 1)
        def _epilogue():
            out_ref[...] = out_ref[...] + x2_s[...]


@jax.jit
def kernel(inputs, lengths, label_len, W1, b1, W2, W3, W0):
    tb_i = lengths[None, :].astype(jnp.int32) - label_len + jnp.arange(LL)[:, None]
    tb_v = tb_i.reshape(LL, B, 1)
    inv = 1.0 / (tb_v.astype(jnp.float32) + 1.0)
    w1t = W1.T.astype(jnp.bfloat16)

    out = pl.pallas_call(
        _body,
        grid=(2, NT),
        in_specs=[
            pl.BlockSpec((TT, B, D), lambda p, i: ((1 - p) * i, 0, 0)),
            pl.BlockSpec((D, H), lambda p, i: (0, 0)),
            pl.BlockSpec((1, H), lambda p, i: (0, 0)),
            pl.BlockSpec((1, H), lambda p, i: (0, 0)),
            pl.BlockSpec((D, H), lambda p, i: (0, 0)),
            pl.BlockSpec((D, H), lambda p, i: (0, 0)),
            pl.BlockSpec((LL, B, 1), lambda p, i: (0, 0, 0)),
            pl.BlockSpec((LL, B, 1), lambda p, i: (0, 0, 0)),
            pl.BlockSpec(memory_space=pltpu.SMEM),
        ],
        out_specs=pl.BlockSpec((LL, B, H), lambda p, i: (0, 0, 0)),
        out_shape=jax.ShapeDtypeStruct((LL, B, H), jnp.float32),
        scratch_shapes=[
            pltpu.VMEM((T * B, D), jnp.bfloat16),
            pltpu.VMEM((B, D), jnp.float32),
            pltpu.VMEM((LL, B, D), jnp.float32),
            pltpu.VMEM((LL, B, H), jnp.float32),
            pltpu.VMEM((LL, B, H), jnp.float32),
        ],
    )(inputs, w1t, b1.reshape(1, H), W0.reshape(1, H),
      W2.T, W3.T, tb_v, inv, tb_i)

    return jnp.transpose(out, (1, 0, 2))


# speedup vs baseline: 11.6150x; 1.1000x over previous
import jax
import jax.numpy as jnp
from jax import lax
from jax.experimental import pallas as pl
from jax.experimental.pallas import tpu as pltpu

T, B, D, H, LL = 2048, 8, 512, 512, 4
TT = 256
NT = T // TT


def _body(x_ref, w1t_ref, b1_ref, w0_ref, w2t_ref, w3t_ref, tb_v_ref,
          inv_ref, tb_s_ref, out_ref, xc_s, sfull_s, g_s, c_s, x2_s):
    p = pl.program_id(0)
    i = pl.program_id(1)
    start = i * TT
    t_max = tb_s_ref[LL - 1, 0]
    live = start <= t_max

    @pl.when(p == 0)
    def _phase0():
        @pl.when(i == 0)
        def _init():
            sfull_s[...] = jnp.zeros_like(sfull_s)
            g_s[...] = jnp.zeros_like(g_s)

        @pl.when(live)
        def _accum():
            x = x_ref[...]
            xc_s[pl.ds(start * B, TT * B)] = x.astype(jnp.bfloat16).reshape(TT * B, D)
            sfull_s[...] = sfull_s[...] + jnp.sum(x, axis=0)
            for j in range(LL):
                for b in range(B):
                    tjb = tb_s_ref[j, b]

                    @pl.when(jnp.logical_and(tjb >= start,
                                             tjb < start + TT))
                    def _g():
                        g_s[pl.ds(j, 1), pl.ds(b, 1), :] = (
                            x_ref[pl.ds(tjb - start, 1), pl.ds(b, 1), :])

    @pl.when(p == 1)
    def _phase1():
        @pl.when(i == 0)
        def _prologue():
            g = g_s[...]
            s3 = sfull_s[...][None]
            s2 = s3 - g[3][None]
            s1 = s2 - g[2][None]
            s0 = s1 - g[1][None]
            s_all = jnp.concatenate([s0, s1, s2, s3], axis=0)
            g2 = jnp.dot(g.reshape(LL * B, D), w2t_ref[...],
                         preferred_element_type=jnp.float32)
            sw = jnp.dot(s_all.reshape(LL * B, D), w3t_ref[...],
                         preferred_element_type=jnp.float32)
            inv = inv_ref[...].reshape(LL * B, 1)
            x2_s[...] = g2.reshape(LL, B, H)
            c_s[...] = (g2 + sw * inv).reshape(LL, B, H)
            out_ref[...] = jnp.zeros_like(out_ref)

        @pl.when(live)
        def _attend():
            xb = xc_s[pl.ds(start * B, TT * B)]
            x1 = jnp.dot(xb, w1t_ref[...],
                         preferred_element_type=jnp.float32) + b1_ref[...]
            x1 = x1.reshape(TT, B, H)
            w0c = w0_ref[...].reshape(H, 1).astype(jnp.bfloat16)
            tvec = lax.broadcasted_iota(jnp.int32, (TT, B, 1), 0) + start
            tb = tb_v_ref[...]
            for j in range(LL):
                @pl.when(start <= tb_s_ref[j, 0])
                def _att_j():
                    cj = c_s[j][None]
                    sg = jax.nn.sigmoid(x1 + cj).astype(jnp.bfloat16)
                    score = jnp.dot(sg.reshape(TT * B, H), w0c,
                                    preferred_element_type=jnp.float32)
                    score = score.reshape(TT, B, 1)
                    le = (tvec <= tb[j][None]).astype(jnp.float32)
                    out_ref[j] = out_ref[j] + jnp.sum(x1 * (score * le),
                                                      axis=0)

        @pl.when(i == NT - 1)
        def _epilogue():
            out_ref[...] = out_ref[...] + x2_s[...]


@jax.jit
def kernel(inputs, lengths, label_len, W1, b1, W2, W3, W0):
    tb_i = lengths[None, :].astype(jnp.int32) - label_len + jnp.arange(LL)[:, None]
    tb_v = tb_i.reshape(LL, B, 1)
    inv = 1.0 / (tb_v.astype(jnp.float32) + 1.0)
    w1t = W1.T.astype(jnp.bfloat16)

    out = pl.pallas_call(
        _body,
        grid=(2, NT),
        in_specs=[
            pl.BlockSpec((TT, B, D), lambda p, i: ((1 - p) * i, 0, 0)),
            pl.BlockSpec((D, H), lambda p, i: (0, 0)),
            pl.BlockSpec((1, H), lambda p, i: (0, 0)),
            pl.BlockSpec((1, H), lambda p, i: (0, 0)),
            pl.BlockSpec((D, H), lambda p, i: (0, 0)),
            pl.BlockSpec((D, H), lambda p, i: (0, 0)),
            pl.BlockSpec((LL, B, 1), lambda p, i: (0, 0, 0)),
            pl.BlockSpec((LL, B, 1), lambda p, i: (0, 0, 0)),
            pl.BlockSpec(memory_space=pltpu.SMEM),
        ],
        out_specs=pl.BlockSpec((LL, B, H), lambda p, i: (0, 0, 0)),
        out_shape=jax.ShapeDtypeStruct((LL, B, H), jnp.float32),
        scratch_shapes=[
            pltpu.VMEM((T * B, D), jnp.bfloat16),
            pltpu.VMEM((B, D), jnp.float32),
            pltpu.VMEM((LL, B, D), jnp.float32),
            pltpu.VMEM((LL, B, H), jnp.float32),
            pltpu.VMEM((LL, B, H), jnp.float32),
        ],
    )(inputs, w1t, b1.reshape(1, H), W0.reshape(1, H),
      W2.T, W3.T, tb_v, inv, tb_i)

    return jnp.transpose(out, (1, 0, 2))


# speedup vs baseline: 13.2196x; 1.1381x over previous
import jax
import jax.numpy as jnp
from jax import lax
from jax.experimental import pallas as pl
from jax.experimental.pallas import tpu as pltpu

T, B, D, H, LL = 2048, 8, 512, 512, 4
TT = 256
NT = T // TT


def _body(x_ref, w1t_ref, b1_ref, w0_ref, w2t_ref, w3t_ref, tb_v_ref,
          inv_ref, tb_s_ref, out_ref, xc_s, sfull_s, g_s, c_s, x2_s,
          w0sum_s):
    p = pl.program_id(0)
    i = pl.program_id(1)
    start = i * TT
    t_max = tb_s_ref[LL - 1, 0]
    live = start <= t_max

    @pl.when(p == 0)
    def _phase0():
        @pl.when(i == 0)
        def _init():
            sfull_s[...] = jnp.zeros_like(sfull_s)
            g_s[...] = jnp.zeros_like(g_s)

        @pl.when(live)
        def _accum():
            x = x_ref[...]
            xc_s[pl.ds(start * B, TT * B)] = x.astype(jnp.bfloat16).reshape(TT * B, D)
            sfull_s[...] = sfull_s[...] + jnp.sum(x, axis=0)
            for j in range(LL):
                for b in range(B):
                    tjb = tb_s_ref[j, b]

                    @pl.when(jnp.logical_and(tjb >= start,
                                             tjb < start + TT))
                    def _g():
                        g_s[pl.ds(j, 1), pl.ds(b, 1), :] = (
                            x_ref[pl.ds(tjb - start, 1), pl.ds(b, 1), :])

    @pl.when(p == 1)
    def _phase1():
        @pl.when(i == 0)
        def _prologue():
            g = g_s[...]
            s3 = sfull_s[...][None]
            s2 = s3 - g[3][None]
            s1 = s2 - g[2][None]
            s0 = s1 - g[1][None]
            s_all = jnp.concatenate([s0, s1, s2, s3], axis=0)
            g2 = jnp.dot(g.reshape(LL * B, D), w2t_ref[...],
                         preferred_element_type=jnp.float32)
            sw = jnp.dot(s_all.reshape(LL * B, D), w3t_ref[...],
                         preferred_element_type=jnp.float32)
            inv = inv_ref[...].reshape(LL * B, 1)
            x2_s[...] = g2.reshape(LL, B, H)
            c_s[...] = (0.5 * (g2 + sw * inv)).reshape(LL, B, H)
            w0sum_s[0, 0] = 0.5 * jnp.sum(w0_ref[...])
            out_ref[...] = jnp.zeros_like(out_ref)

        @pl.when(live)
        def _attend():
            xb = xc_s[pl.ds(start * B, TT * B)]
            x1 = jnp.dot(xb, w1t_ref[...],
                         preferred_element_type=jnp.float32) + b1_ref[...]
            x1 = x1.reshape(TT, B, H)
            x1h = 0.5 * x1
            w0c = (0.5 * w0_ref[...]).reshape(H, 1).astype(jnp.bfloat16)
            w0sum = w0sum_s[0, 0]
            tvec = lax.broadcasted_iota(jnp.int32, (TT, B, 1), 0) + start
            tb = tb_v_ref[...]
            for j in range(LL):
                @pl.when(start <= tb_s_ref[j, 0])
                def _att_j():
                    cj = c_s[j][None]
                    th = jnp.tanh(x1h + cj).astype(jnp.bfloat16)
                    score = w0sum + jnp.dot(th.reshape(TT * B, H), w0c,
                                            preferred_element_type=jnp.float32)
                    score = score.reshape(TT, B, 1)
                    le = (tvec <= tb[j][None]).astype(jnp.float32)
                    out_ref[j] = out_ref[j] + jnp.sum(x1 * (score * le),
                                                      axis=0)

        @pl.when(i == NT - 1)
        def _epilogue():
            out_ref[...] = out_ref[...] + x2_s[...]


@jax.jit
def kernel(inputs, lengths, label_len, W1, b1, W2, W3, W0):
    tb_i = lengths[None, :].astype(jnp.int32) - label_len + jnp.arange(LL)[:, None]
    tb_v = tb_i.reshape(LL, B, 1)
    inv = 1.0 / (tb_v.astype(jnp.float32) + 1.0)
    w1t = W1.T.astype(jnp.bfloat16)

    out = pl.pallas_call(
        _body,
        grid=(2, NT),
        in_specs=[
            pl.BlockSpec((TT, B, D), lambda p, i: ((1 - p) * i, 0, 0)),
            pl.BlockSpec((D, H), lambda p, i: (0, 0)),
            pl.BlockSpec((1, H), lambda p, i: (0, 0)),
            pl.BlockSpec((1, H), lambda p, i: (0, 0)),
            pl.BlockSpec((D, H), lambda p, i: (0, 0)),
            pl.BlockSpec((D, H), lambda p, i: (0, 0)),
            pl.BlockSpec((LL, B, 1), lambda p, i: (0, 0, 0)),
            pl.BlockSpec((LL, B, 1), lambda p, i: (0, 0, 0)),
            pl.BlockSpec(memory_space=pltpu.SMEM),
        ],
        out_specs=pl.BlockSpec((LL, B, H), lambda p, i: (0, 0, 0)),
        out_shape=jax.ShapeDtypeStruct((LL, B, H), jnp.float32),
        scratch_shapes=[
            pltpu.VMEM((T * B, D), jnp.bfloat16),
            pltpu.VMEM((B, D), jnp.float32),
            pltpu.VMEM((LL, B, D), jnp.float32),
            pltpu.VMEM((LL, B, H), jnp.float32),
            pltpu.VMEM((LL, B, H), jnp.float32),
            pltpu.SMEM((1, 1), jnp.float32),
        ],
    )(inputs, w1t, b1.reshape(1, H), W0.reshape(1, H),
      W2.T, W3.T, tb_v, inv, tb_i)

    return jnp.transpose(out, (1, 0, 2))
